# baseline (speedup 1.0000x reference)
import functools

import jax
import jax.numpy as jnp
from jax import lax
from jax.experimental import pallas as pl
from jax.experimental.pallas import tpu as pltpu
from jax.experimental.pallas import tpu_sc as plsc

_SF = 2
_D = 28 * _SF
_DX = 172 * _SF
_DOUT = 3 * _D + _DX
_NTAB = 2112
_TW = 256
_BB = 128
_NC, _NS = 2, 16
_NW = _NC * _NS
_CHUNK = 256


def _prep_body(xt_ref, wt_ref, idx_ref, tab_ref):
    xt = xt_ref[...]
    idx = 105.0 * xt[0] + 21.0 * xt[1] + xt[2]
    idx_ref[...] = idx.astype(jnp.int32)
    r = lax.broadcasted_iota(jnp.int32, (_NTAB, 1), 0)
    t1 = r // 105
    t2 = (r // 21) % 5 + 20
    t3 = r % 21 + 25
    c = lax.broadcasted_iota(jnp.int32, (_NTAB, 64), 1)
    oh = ((c == t1) | (c == t2) | (c == t3)).astype(jnp.float32)
    tab_ref[...] = lax.dot_general(oh, wt_ref[...], (((1,), (0,)), ((), ())),
                                   preferred_element_type=jnp.float32)


def _make_sc_gather(n):
    per_w = n // _NW
    mesh = plsc.VectorSubcoreMesh(core_axis_name="c", subcore_axis_name="s")

    @functools.partial(
        pl.kernel,
        mesh=mesh,
        out_type=jax.ShapeDtypeStruct((n, _TW), jnp.float32),
        scratch_types=[
            pltpu.VMEM((_CHUNK,), jnp.int32),
            pltpu.VMEM((_CHUNK, _TW), jnp.float32),
            pltpu.SemaphoreType.DMA,
        ],
    )
    def sc_gather(tab_hbm, idx_hbm, out_hbm, idx_v, rows_v, sem):
        wid = lax.axis_index("s") * _NC + lax.axis_index("c")
        base = wid * per_w

        @pl.loop(0, per_w // _CHUNK)
        def _(it):
            off = base + it * _CHUNK
            pltpu.sync_copy(idx_hbm.at[pl.ds(off, _CHUNK)], idx_v)
            pltpu.async_copy(tab_hbm.at[idx_v], rows_v, sem).wait()
            pltpu.sync_copy(rows_v, out_hbm.at[pl.ds(off, _CHUNK)])

    return sc_gather


def _merge_body(xt_ref, wm_ref, g_ref, out_ref):
    xt = xt_ref[...]
    _, l, bb = xt.shape
    j = pl.program_id(1)
    c = lax.broadcasted_iota(jnp.int32, (8, l, bb), 0)
    feats = (c == 3).astype(jnp.float32)
    feats += jnp.where(c == 0, xt[3:4], 0.0)
    feats += jnp.where(c == 1, xt[4:5], 0.0)
    feats += jnp.where(c == 2, xt[5:6], 0.0)
    y = lax.dot_general(feats, wm_ref[...], (((0,), (0,)), ((), ())),
                        preferred_element_type=jnp.float32)
    y += g_ref[...] * jnp.where(j == 0, 1.0, 0.0)
    out_ref[...] = jnp.maximum(y, wm_ref[4:5, :][None])


def _pack_weights(amino_table, element_table, position_table, W_xyz, b_xyz):
    wt = jnp.zeros((64, _TW), dtype=jnp.float32)
    wt = wt.at[0:20, 0:_D].set(amino_table)
    wt = wt.at[20:25, _D:2 * _D].set(element_table)
    wt = wt.at[25:46, 2 * _D:3 * _D].set(position_table)
    wm = jnp.zeros((8, _DOUT), dtype=jnp.float32)
    wm = wm.at[0:3, 3 * _D:].set(W_xyz)
    wm = wm.at[3, 3 * _D:].set(b_xyz)
    wm = wm.at[4, 0:3 * _D].set(jnp.finfo(jnp.float32).min)
    return wt, wm


def kernel(x, amino_table, element_table, position_table, W_xyz, b_xyz):
    B, L, _ = x.shape
    n = B * L
    wt, wm = _pack_weights(amino_table, element_table, position_table, W_xyz,
                           b_xyz)
    xt = jnp.transpose(x, (2, 1, 0))

    idx2d, tab = pl.pallas_call(
        _prep_body,
        grid=(1,),
        in_specs=[
            pl.BlockSpec((6, L, B), lambda i: (0, 0, 0)),
            pl.BlockSpec((64, _TW), lambda i: (0, 0)),
        ],
        out_specs=[
            pl.BlockSpec((L, B), lambda i: (0, 0)),
            pl.BlockSpec((_NTAB, _TW), lambda i: (0, 0)),
        ],
        out_shape=[
            jax.ShapeDtypeStruct((L, B), jnp.int32),
            jax.ShapeDtypeStruct((_NTAB, _TW), jnp.float32),
        ],
    )(xt, wt)

    g = _make_sc_gather(n)(tab, idx2d.reshape(n))
    gv = g.reshape(L, B, _TW)

    out_t = pl.pallas_call(
        _merge_body,
        grid=(B // _BB, 2),
        in_specs=[
            pl.BlockSpec((6, L, _BB), lambda i, j: (0, 0, i)),
            pl.BlockSpec((8, _DOUT // 2), lambda i, j: (0, j)),
            pl.BlockSpec((L, _BB, _TW), lambda i, j: (0, i, 0)),
        ],
        out_specs=pl.BlockSpec((L, _BB, _DOUT // 2), lambda i, j: (0, i, j)),
        out_shape=jax.ShapeDtypeStruct((L, B, _DOUT), jnp.float32),
        compiler_params=pltpu.CompilerParams(
            dimension_semantics=("parallel", "arbitrary"),
        ),
    )(xt, wm, gv)
    return jnp.transpose(out_t, (1, 0, 2))


# speedup vs baseline: 3.2608x; 3.2608x over previous
import jax
import jax.numpy as jnp
from jax import lax
from jax.experimental import pallas as pl
from jax.experimental.pallas import tpu as pltpu

_SF = 2
_D = 28 * _SF
_DX = 172 * _SF
_DOUT = 3 * _D + _DX
_K = 64
_BB = 128


def _fused_body(xt_ref, w_ref, out_ref):
    xt = xt_ref[...]
    _, l, bb = xt.shape
    t1 = xt[0:1].astype(jnp.int32)
    t2 = xt[1:2].astype(jnp.int32) + 20
    t3 = xt[2:3].astype(jnp.int32) + 25

    c = lax.broadcasted_iota(jnp.int32, (_K, l, bb), 0)
    ones = (c == t1) | (c == t2) | (c == t3) | (c == 49)
    feats = ones.astype(jnp.float32)
    feats += jnp.where(c == 46, xt[3:4], 0.0)
    feats += jnp.where(c == 47, xt[4:5], 0.0)
    feats += jnp.where(c == 48, xt[5:6], 0.0)

    y = lax.dot_general(feats, w_ref[...], (((0,), (0,)), ((), ())),
                        preferred_element_type=jnp.float32)
    out_ref[...] = jnp.maximum(y, w_ref[50:51, :][None])


def _pack_weights(amino_table, element_table, position_table, W_xyz, b_xyz):
    w = jnp.zeros((_K, _DOUT), dtype=jnp.float32)
    w = w.at[0:20, 0:_D].set(amino_table)
    w = w.at[20:25, _D:2 * _D].set(element_table)
    w = w.at[25:46, 2 * _D:3 * _D].set(position_table)
    w = w.at[46:49, 3 * _D:].set(W_xyz)
    w = w.at[49, 3 * _D:].set(b_xyz)
    w = w.at[50, 0:3 * _D].set(jnp.finfo(jnp.float32).min)
    return w


def kernel(x, amino_table, element_table, position_table, W_xyz, b_xyz):
    B, L, _ = x.shape
    w = _pack_weights(amino_table, element_table, position_table, W_xyz,
                      b_xyz)
    xt = jnp.transpose(x, (2, 1, 0))
    out_t = pl.pallas_call(
        _fused_body,
        grid=(B // _BB,),
        in_specs=[
            pl.BlockSpec((6, L, _BB), lambda i: (0, 0, i)),
            pl.BlockSpec((_K, _DOUT), lambda i: (0, 0)),
        ],
        out_specs=pl.BlockSpec((L, _BB, _DOUT), lambda i: (0, i, 0)),
        out_shape=jax.ShapeDtypeStruct((L, B, _DOUT), jnp.float32),
        compiler_params=pltpu.CompilerParams(
            dimension_semantics=("parallel",),
        ),
    )(xt, w)
    return jnp.transpose(out_t, (1, 0, 2))
